# baseline (speedup 1.0000x reference)
import math
from functools import partial

import jax
import jax.numpy as jnp
from jax import lax
from jax.experimental import pallas as pl
from jax.experimental.pallas import tpu as pltpu

_HIDDEN = 16
_NUM_HEADS = 2
_HEAD_DIM = _HIDDEN // _NUM_HEADS
_LN_EPS = 1e-12


def _attn_kernel(x_ref, w_ref, vec_ref, out_ref, *,
                 G, S, H, num_heads, head_dim):
    x2 = x_ref[...]
    w = w_ref[...]
    vec = vec_ref[...]

    proj = jnp.dot(x2, w, preferred_element_type=jnp.float32) + vec[0:1, 0:4 * H]

    def score_pair(g):
        pg = proj[g * S:(g + 1) * S, :]
        out = []
        for h in range(num_heads):
            lo = h * head_dim
            q = pg[:, lo:lo + head_dim]
            k = pg[:, H + lo:H + lo + head_dim]
            out.append(
                lax.dot_general(q, k, (((1,), (1,)), ((), ())),
                                preferred_element_type=jnp.float32))
        return out

    def softmax_pair(sh):
        m = sh[0]
        for s in sh[1:]:
            m = jnp.maximum(m, s)
        m = jnp.max(m, axis=-1, keepdims=True)
        out = []
        for s in sh:
            e = jnp.exp2(s - m)
            out.append(e * pl.reciprocal(
                jnp.sum(e, axis=-1, keepdims=True), approx=True))
        return out

    def make_vo(g):
        pg = proj[g * S:(g + 1) * S, :]
        return jnp.concatenate(
            [pg[:, 2 * H + h * H:2 * H + (h + 1) * H] for h in range(num_heads)],
            axis=0)

    def ctx(g, pr, vo):
        p_cat = jnp.concatenate(pr, axis=1)
        return jnp.dot(p_cat, vo, preferred_element_type=jnp.float32)

    def tail(c, parts, n_chunks):
        rows = G * S // n_chunks
        r0 = c * rows
        y = jnp.concatenate(parts, axis=0) \
            + x2[r0:r0 + rows, :] + vec[0:1, 4 * H:5 * H]
        mean = jnp.mean(y, axis=-1, keepdims=True)
        mean_sq = jnp.mean(y * y, axis=-1, keepdims=True)
        var = mean_sq - mean * mean
        out = (y - mean) * lax.rsqrt(var + _LN_EPS) * vec[0:1, 5 * H:6 * H] \
            + vec[0:1, 6 * H:7 * H]
        out_ref[r0:r0 + rows, :] = out.astype(out_ref.dtype)

    sc = [score_pair(g) for g in range(G)]
    vo = [make_vo(g) for g in range(G)]
    pr = [softmax_pair(sc[g]) for g in range(G)]
    yp = [ctx(g, pr[g], vo[g]) for g in range(G)]
    tail(0, yp, 1)


def kernel(hidden_states, wq, bq, wk, bk, wv, bv, wo, bo, gamma, beta):
    B, S, H = hidden_states.shape
    nh = _NUM_HEADS
    hd = H // nh
    scale = math.log2(math.e) / math.sqrt(hd)

    wo_t = wo.T
    wvo = [wv.T[:, h * hd:(h + 1) * hd] @ wo_t[h * hd:(h + 1) * hd, :]
           for h in range(nh)]
    bvo = [bv[h * hd:(h + 1) * hd] @ wo_t[h * hd:(h + 1) * hd, :]
           for h in range(nh)]

    w_pack = jnp.concatenate([wq.T * scale, wk.T] + wvo, axis=1)
    vec_pack = jnp.concatenate(
        [bq * scale, bk] + bvo + [bo, gamma, beta])[None, :]

    G = next(g for g in (8, 4, 2, 1) if B % g == 0)

    kfn = partial(_attn_kernel, G=G, S=S, H=H, num_heads=nh, head_dim=hd)

    x2d = hidden_states.reshape(B * S, H)

    out = pl.pallas_call(
        kfn,
        out_shape=jax.ShapeDtypeStruct((B * S, H), hidden_states.dtype),
        grid=(B // G,),
        in_specs=[
            pl.BlockSpec((G * S, H), lambda b: (b, 0)),
            pl.BlockSpec(w_pack.shape, lambda b: (0, 0)),
            pl.BlockSpec(vec_pack.shape, lambda b: (0, 0)),
        ],
        out_specs=pl.BlockSpec((G * S, H), lambda b: (b, 0)),
        compiler_params=pltpu.CompilerParams(
            dimension_semantics=("arbitrary",)),
    )(x2d, w_pack, vec_pack)

    return out.reshape(B, S, H)


# speedup vs baseline: 1.0440x; 1.0440x over previous
import math
from functools import partial

import jax
import jax.numpy as jnp
from jax import lax
from jax.experimental import pallas as pl
from jax.experimental.pallas import tpu as pltpu

_HIDDEN = 16
_NUM_HEADS = 2
_HEAD_DIM = _HIDDEN // _NUM_HEADS
_LN_EPS = 1e-12


def _attn_kernel(x_ref, w_ref, vec_ref, out_ref, *,
                 G, S, H, num_heads, head_dim, halves):
    w = w_ref[...]
    vec = vec_ref[...]

    for half in range(halves):
        _attn_half(x_ref, vec, w, out_ref, half, G=G, S=S, H=H,
                   num_heads=num_heads, head_dim=head_dim)


def _attn_half(x_ref, vec, w, out_ref, half, *, G, S, H, num_heads, head_dim):
    base = half * G * S
    x2 = x_ref[base:base + G * S, :]

    proj = jnp.dot(x2, w, preferred_element_type=jnp.float32) + vec[0:1, 0:4 * H]

    def score_pair(g):
        pg = proj[g * S:(g + 1) * S, :]
        out = []
        for h in range(num_heads):
            lo = h * head_dim
            q = pg[:, lo:lo + head_dim]
            k = pg[:, H + lo:H + lo + head_dim]
            out.append(
                lax.dot_general(q, k, (((1,), (1,)), ((), ())),
                                preferred_element_type=jnp.float32))
        return out

    def softmax_pair(sh):
        m = sh[0]
        for s in sh[1:]:
            m = jnp.maximum(m, s)
        m = jnp.max(m, axis=-1, keepdims=True)
        out = []
        for s in sh:
            e = jnp.exp2(s - m)
            out.append(e * pl.reciprocal(
                jnp.sum(e, axis=-1, keepdims=True), approx=True))
        return out

    def make_vo(g):
        pg = proj[g * S:(g + 1) * S, :]
        return jnp.concatenate(
            [pg[:, 2 * H + h * H:2 * H + (h + 1) * H] for h in range(num_heads)],
            axis=0)

    def ctx(g, pr, vo):
        p_cat = jnp.concatenate(pr, axis=1)
        return jnp.dot(p_cat, vo, preferred_element_type=jnp.float32)

    def tail(c, parts, n_chunks):
        rows = G * S // n_chunks
        r0 = c * rows
        y = jnp.concatenate(parts, axis=0) \
            + x2[r0:r0 + rows, :] + vec[0:1, 4 * H:5 * H]
        mean = jnp.mean(y, axis=-1, keepdims=True)
        mean_sq = jnp.mean(y * y, axis=-1, keepdims=True)
        var = mean_sq - mean * mean
        out = (y - mean) * lax.rsqrt(var + _LN_EPS) * vec[0:1, 5 * H:6 * H] \
            + vec[0:1, 6 * H:7 * H]
        out_ref[base + r0:base + r0 + rows, :] = out.astype(out_ref.dtype)

    sc = [score_pair(g) for g in range(G)]
    vo = [make_vo(g) for g in range(G)]
    pr = [softmax_pair(sc[g]) for g in range(G)]
    yp = [ctx(g, pr[g], vo[g]) for g in range(G)]
    tail(0, yp, 1)


def kernel(hidden_states, wq, bq, wk, bk, wv, bv, wo, bo, gamma, beta):
    B, S, H = hidden_states.shape
    nh = _NUM_HEADS
    hd = H // nh
    scale = math.log2(math.e) / math.sqrt(hd)

    wo_t = wo.T
    wvo = [wv.T[:, h * hd:(h + 1) * hd] @ wo_t[h * hd:(h + 1) * hd, :]
           for h in range(nh)]
    bvo = [bv[h * hd:(h + 1) * hd] @ wo_t[h * hd:(h + 1) * hd, :]
           for h in range(nh)]

    w_pack = jnp.concatenate([wq.T * scale, wk.T] + wvo, axis=1)
    vec_pack = jnp.concatenate(
        [bq * scale, bk] + bvo + [bo, gamma, beta])[None, :]

    G = next(g for g in (8, 4, 2, 1) if B % g == 0)
    halves = 2 if B % (2 * G) == 0 else 1

    kfn = partial(_attn_kernel, G=G, S=S, H=H, num_heads=nh, head_dim=hd,
                  halves=halves)

    x2d = hidden_states.reshape(B * S, H)

    out = pl.pallas_call(
        kfn,
        out_shape=jax.ShapeDtypeStruct((B * S, H), hidden_states.dtype),
        grid=(B // (halves * G),),
        in_specs=[
            pl.BlockSpec((halves * G * S, H), lambda b: (b, 0)),
            pl.BlockSpec(w_pack.shape, lambda b: (0, 0)),
            pl.BlockSpec(vec_pack.shape, lambda b: (0, 0)),
        ],
        out_specs=pl.BlockSpec((halves * G * S, H), lambda b: (b, 0)),
        compiler_params=pltpu.CompilerParams(
            dimension_semantics=("parallel",)),
    )(x2d, w_pack, vec_pack)

    return out.reshape(B, S, H)


# speedup vs baseline: 1.0698x; 1.0247x over previous
import math
from functools import partial

import jax
import jax.numpy as jnp
from jax import lax
from jax.experimental import pallas as pl
from jax.experimental.pallas import tpu as pltpu

_HIDDEN = 16
_NUM_HEADS = 2
_HEAD_DIM = _HIDDEN // _NUM_HEADS
_LN_EPS = 1e-12


def _attn_kernel(x_ref, w_ref, vec_ref, out_ref, *,
                 G, S, H, num_heads, head_dim, halves):
    w = w_ref[...]
    vec = vec_ref[...]

    for half in range(halves):
        _attn_half(x_ref, vec, w, out_ref, half, G=G, S=S, H=H,
                   num_heads=num_heads, head_dim=head_dim)


def _attn_half(x_ref, vec, w, out_ref, half, *, G, S, H, num_heads, head_dim):
    base = half * G * S
    x2 = x_ref[base:base + G * S, :]

    proj = jnp.dot(x2, w, preferred_element_type=jnp.float32) + vec[0:1, 0:4 * H]

    def score_pair(g):
        pg = proj[g * S:(g + 1) * S, :]
        out = []
        for h in range(num_heads):
            lo = h * head_dim
            q = pg[:, lo:lo + head_dim]
            k = pg[:, H + lo:H + lo + head_dim]
            out.append(
                lax.dot_general(q, k, (((1,), (1,)), ((), ())),
                                preferred_element_type=jnp.float32))
        return out

    def softmax_pair(sh):
        m = sh[0]
        for s in sh[1:]:
            m = jnp.maximum(m, s)
        m = jnp.max(m, axis=-1, keepdims=True)
        out = []
        for s in sh:
            e = jnp.exp2(s - m)
            out.append(e * pl.reciprocal(
                jnp.sum(e, axis=-1, keepdims=True), approx=True))
        return out

    def make_vo(g):
        pg = proj[g * S:(g + 1) * S, :]
        return jnp.concatenate(
            [pg[:, 2 * H + h * H:2 * H + (h + 1) * H] for h in range(num_heads)],
            axis=0)

    def ctx(g, pr, vo):
        p_cat = jnp.concatenate(pr, axis=1)
        return jnp.dot(p_cat, vo, preferred_element_type=jnp.float32)

    def tail(c, parts, n_chunks):
        rows = G * S // n_chunks
        r0 = c * rows
        y = jnp.concatenate(parts, axis=0) \
            + x2[r0:r0 + rows, :] + vec[0:1, 4 * H:5 * H]
        mean = jnp.mean(y, axis=-1, keepdims=True)
        mean_sq = jnp.mean(y * y, axis=-1, keepdims=True)
        var = mean_sq - mean * mean
        out = (y - mean) * lax.rsqrt(var + _LN_EPS) * vec[0:1, 5 * H:6 * H] \
            + vec[0:1, 6 * H:7 * H]
        out_ref[base + r0:base + r0 + rows, :] = out.astype(out_ref.dtype)

    sc = [score_pair(g) for g in range(G)]
    vo = [make_vo(g) for g in range(G)]
    pr = [softmax_pair(sc[g]) for g in range(G)]
    yp = [ctx(g, pr[g], vo[g]) for g in range(G)]
    tail(0, yp, 1)


def kernel(hidden_states, wq, bq, wk, bk, wv, bv, wo, bo, gamma, beta):
    B, S, H = hidden_states.shape
    nh = _NUM_HEADS
    hd = H // nh
    scale = math.log2(math.e) / math.sqrt(hd)

    wo_t = wo.T
    wvo = [wv.T[:, h * hd:(h + 1) * hd] @ wo_t[h * hd:(h + 1) * hd, :]
           for h in range(nh)]
    bvo = [bv[h * hd:(h + 1) * hd] @ wo_t[h * hd:(h + 1) * hd, :]
           for h in range(nh)]

    w_pack = jnp.concatenate([wq.T * scale, wk.T] + wvo, axis=1)
    vec_pack = jnp.concatenate(
        [bq * scale, bk] + bvo + [bo, gamma, beta])[None, :]

    G = next(g for g in (8, 4, 2, 1) if B % g == 0)
    halves = next(hv for hv in (4, 2, 1) if B % (hv * G) == 0)

    kfn = partial(_attn_kernel, G=G, S=S, H=H, num_heads=nh, head_dim=hd,
                  halves=halves)

    x2d = hidden_states.reshape(B * S, H)

    out = pl.pallas_call(
        kfn,
        out_shape=jax.ShapeDtypeStruct((B * S, H), hidden_states.dtype),
        grid=(B // (halves * G),),
        in_specs=[
            pl.BlockSpec((halves * G * S, H), lambda b: (b, 0)),
            pl.BlockSpec(w_pack.shape, lambda b: (0, 0)),
            pl.BlockSpec(vec_pack.shape, lambda b: (0, 0)),
        ],
        out_specs=pl.BlockSpec((halves * G * S, H), lambda b: (b, 0)),
        compiler_params=pltpu.CompilerParams(
            dimension_semantics=("parallel",)),
    )(x2d, w_pack, vec_pack)

    return out.reshape(B, S, H)
